# baseline (speedup 1.0000x reference)
import jax
import jax.numpy as jnp
from jax import lax
from jax.experimental import pallas as pl
from jax.experimental.pallas import tpu as pltpu
from jax.experimental.pallas import tpu_sc as plsc

EMBED = 128
HIDDEN = 1024
BATCH = 16384
NC = 2
NS = 16
NW = NC * NS

NSPLIT = 2
B_SPLIT = BATCH // NSPLIT
B_PER_W = B_SPLIT // NW
CHUNK = 128
NCHUNK = B_PER_W // CHUNK

BM = 4096
BLOCKS_PER_SPLIT = B_SPLIT // BM


def _gather_body(xp, xo, xs, wp, wo, ws, op, oo, osub, idx_v, rows_v, sem):
    wid = lax.axis_index("s") * NC + lax.axis_index("c")
    base = wid * B_PER_W
    for x_hbm, t_hbm, o_hbm in ((xp, wp, op), (xo, wo, oo), (xs, ws, osub)):
        pltpu.sync_copy(x_hbm.at[pl.ds(base, B_PER_W)], idx_v)
        for j in range(NCHUNK):
            pltpu.async_copy(
                t_hbm.at[idx_v.at[pl.ds(j * CHUNK, CHUNK)]],
                rows_v.at[pl.ds(j * CHUNK, CHUNK)],
                sem,
            )
        for j in range(NCHUNK):
            pltpu.make_async_copy(
                t_hbm.at[idx_v.at[pl.ds(j * CHUNK, CHUNK)]],
                rows_v.at[pl.ds(j * CHUNK, CHUNK)],
                sem,
            ).wait()
        pltpu.sync_copy(rows_v, o_hbm.at[pl.ds(base, B_PER_W)])


_h_type = jax.ShapeDtypeStruct((B_SPLIT, EMBED), jnp.float32)

_gather = pl.kernel(
    _gather_body,
    mesh=plsc.VectorSubcoreMesh(core_axis_name="c", subcore_axis_name="s"),
    out_type=(_h_type, _h_type, _h_type),
    scratch_types=[
        pltpu.VMEM((B_PER_W,), jnp.int32),
        pltpu.VMEM((B_PER_W, EMBED), jnp.float32),
        pltpu.SemaphoreType.DMA,
    ],
)


def _mlp_first_body(hp, ho, hs, w, b, o):
    h = jnp.concatenate(
        (
            jnp.maximum(hp[...], 0.0),
            jnp.maximum(ho[...], 0.0),
            jnp.maximum(hs[...], 0.0),
        ),
        axis=1,
    ).astype(jnp.bfloat16)
    acc = jnp.dot(h, w[...], preferred_element_type=jnp.float32)
    o[...] = jnp.maximum(acc + b[...], 0.0)


def _mlp_next_body(hp, ho, hs, w, b, prev, o):
    del prev
    _mlp_first_body(hp, ho, hs, w, b, o)


_OUT_TYPE = jax.ShapeDtypeStruct((BATCH, HIDDEN), jnp.float32)

_H_SPECS = [
    pl.BlockSpec((BM, EMBED), lambda i: (i, 0)),
    pl.BlockSpec((BM, EMBED), lambda i: (i, 0)),
    pl.BlockSpec((BM, EMBED), lambda i: (i, 0)),
    pl.BlockSpec((3 * EMBED, HIDDEN), lambda i: (0, 0)),
    pl.BlockSpec((1, HIDDEN), lambda i: (0, 0)),
]


def _mlp_chunk(split, hp, ho, hs, w, b, prev=None):
    off = split * BLOCKS_PER_SPLIT
    out_spec = pl.BlockSpec((BM, HIDDEN), lambda i: (i + off, 0))
    if prev is None:
        return pl.pallas_call(
            _mlp_first_body,
            grid=(BLOCKS_PER_SPLIT,),
            in_specs=_H_SPECS,
            out_specs=out_spec,
            out_shape=_OUT_TYPE,
        )(hp, ho, hs, w, b)
    return pl.pallas_call(
        _mlp_next_body,
        grid=(BLOCKS_PER_SPLIT,),
        in_specs=_H_SPECS + [pl.BlockSpec(memory_space=pl.ANY)],
        out_specs=out_spec,
        out_shape=_OUT_TYPE,
        input_output_aliases={5: 0},
    )(hp, ho, hs, w, b, prev)


def kernel(X_phase, X_occurrence, X_subject, X_lengths,
           W_phase, W_occurrence, W_subject, W_out, b_out):
    del X_lengths
    xp = X_phase.astype(jnp.int32)
    xo = X_occurrence.astype(jnp.int32)
    xs = X_subject.astype(jnp.int32)
    w_bf = W_out.astype(jnp.bfloat16)
    b2d = b_out.reshape(1, HIDDEN)

    h_chunks = []
    for c in range(NSPLIT):
        sl = pl.ds(c * B_SPLIT, B_SPLIT)
        h_chunks.append(
            _gather(xp[sl], xo[sl], xs[sl], W_phase, W_occurrence, W_subject)
        )
    out = None
    for c, (hp, ho, hs) in enumerate(h_chunks):
        out = _mlp_chunk(c, hp, ho, hs, w_bf, b2d, out)
    return out
